# baseline (speedup 1.0000x reference)
import functools

import jax
import jax.numpy as jnp
from jax import lax
from jax.experimental import pallas as pl
from jax.experimental.pallas import tpu as pltpu
from jax.experimental.pallas import tpu_sc as plsc

BATCH = 16384
EMBED = 32
NUM_CORES = 2
NUM_SUBCORES = 16
NUM_WORKERS = NUM_CORES * NUM_SUBCORES
B_PER_W = BATCH // NUM_WORKERS
CHUNK = 128
NCHUNK = B_PER_W // CHUNK
LANES = 16
NGROUP = B_PER_W // LANES


_PERM_DNUMS = lax.GatherDimensionNumbers(
    offset_dims=(), collapsed_slice_dims=(0,), start_index_map=(0,))


def _lane_perm(v, idx):
    return lax.gather(v, idx[:, None], _PERM_DNUMS, (1,),
                      unique_indices=True, indices_are_sorted=False,
                      mode=lax.GatherScatterMode.PROMISE_IN_BOUNDS)


@functools.partial(
    pl.kernel,
    mesh=plsc.VectorSubcoreMesh(core_axis_name="c", subcore_axis_name="s"),
    out_type=jax.ShapeDtypeStruct((BATCH,), jnp.float32),
    compiler_params=pltpu.CompilerParams(use_tc_tiling_on_sc=False),
    scratch_types=[
        pltpu.VMEM((B_PER_W,), jnp.int32),
        pltpu.VMEM((B_PER_W,), jnp.int32),
        pltpu.VMEM((B_PER_W, EMBED), jnp.float32),
        pltpu.VMEM((B_PER_W, EMBED), jnp.float32),
        pltpu.VMEM((B_PER_W,), jnp.float32),
        pltpu.VMEM((B_PER_W,), jnp.float32),
        pltpu.VMEM((B_PER_W,), jnp.float32),
        pltpu.SemaphoreType.DMA,
    ],
)
def _mf_sc(user_hbm, item_hbm, ue_hbm, ie_hbm, ub_hbm, ib_hbm, out_hbm,
           uidx_v, iidx_v, ue_v, ie_v, ub_v, ib_v, res_v, sem):
    wid = lax.axis_index("s") * NUM_CORES + lax.axis_index("c")
    base = wid * B_PER_W

    pltpu.sync_copy(user_hbm.at[pl.ds(base, B_PER_W)], uidx_v)
    pltpu.sync_copy(item_hbm.at[pl.ds(base, B_PER_W)], iidx_v)

    copies = []
    for j in range(NCHUNK):
        sl = pl.ds(j * CHUNK, CHUNK)
        copies.append(pltpu.async_copy(ue_hbm.at[uidx_v.at[sl]], ue_v.at[sl], sem))
        copies.append(pltpu.async_copy(ie_hbm.at[iidx_v.at[sl]], ie_v.at[sl], sem))
        copies.append(pltpu.async_copy(ub_hbm.at[uidx_v.at[sl]], ub_v.at[sl], sem))
        copies.append(pltpu.async_copy(ib_hbm.at[iidx_v.at[sl]], ib_v.at[sl], sem))
    for c in copies:
        c.wait()

    iota = lax.iota(jnp.int32, LANES)
    perms = {k: iota ^ k for k in (8, 4, 2, 1)}
    masks = {k: (iota & k) == 0 for k in (8, 4, 2, 1)}

    def group_body(g, carry):
        r0 = g * LANES
        vs = []
        for t in range(LANES):
            r = r0 + t
            u0 = ue_v[r, pl.ds(0, LANES)]
            u1 = ue_v[r, pl.ds(LANES, LANES)]
            i0 = ie_v[r, pl.ds(0, LANES)]
            i1 = ie_v[r, pl.ds(LANES, LANES)]
            vs.append(u0 * i0 + u1 * i1)
        cur = vs
        for k in (8, 4, 2, 1):
            nxt = []
            for i in range(k):
                fa = cur[i] + _lane_perm(cur[i], perms[k])
                fb = cur[i + k] + _lane_perm(cur[i + k], perms[k])
                nxt.append(jnp.where(masks[k], fa, fb))
            cur = nxt
        dots = cur[0]
        tot = dots + ub_v[pl.ds(r0, LANES)] + ib_v[pl.ds(r0, LANES)]
        res_v[pl.ds(r0, LANES)] = 1.0 / (1.0 + jnp.exp(-tot))
        return carry

    lax.fori_loop(0, NGROUP, group_body, 0)

    pltpu.sync_copy(res_v, out_hbm.at[pl.ds(base, B_PER_W)])


def kernel(user, item, user_e, item_e, user_b, item_b):
    return _mf_sc(user, item, user_e, item_e,
                  user_b.reshape(-1), item_b.reshape(-1))
